# baseline (device time: 28602 ns/iter reference)
import jax
import jax.numpy as jnp
from jax import lax
from jax.experimental import pallas as pl
from jax.experimental.pallas import tpu as pltpu

N_DEV = 32
N_TOK = 256
D_IN = 128
D_OUT = 256
EXP_PER = 2
CAPACITY = 3
N_STEPS = 5


def kernel(x, router_W, route_idx, expert_W):
    del router_W

    def body(x_ref, ridx_ref, ew_ref, out_ref,
             acc_ref, send_ref, recv_ref, send_sems, recv_sems):
        my = lax.axis_index("i")

        barrier_sem = pltpu.get_barrier_semaphore()
        for k in range(N_STEPS):
            pl.semaphore_signal(
                barrier_sem, inc=1,
                device_id=(my ^ (1 << k),),
                device_id_type=pl.DeviceIdType.MESH,
            )
        pl.semaphore_wait(barrier_sem, N_STEPS)

        ridx = ridx_ref[...]
        e0 = my * EXP_PER
        m0 = (ridx == e0).astype(jnp.float32)
        m1 = (ridx == e0 + 1).astype(jnp.float32)
        m = jnp.concatenate([m0, m1], axis=1)
        row = lax.broadcasted_iota(jnp.int32, (N_TOK, N_TOK), 0)
        col = lax.broadcasted_iota(jnp.int32, (N_TOK, N_TOK), 1)
        tri = (row >= col).astype(jnp.float32)
        pos = jnp.dot(tri, m, preferred_element_type=jnp.float32)
        keep = m * (pos <= float(CAPACITY))

        xb = x_ref[...].astype(jnp.bfloat16)
        w = ew_ref[...]
        y = jnp.dot(xb * keep[:, 0:1].astype(jnp.bfloat16),
                    w[0].astype(jnp.bfloat16),
                    preferred_element_type=jnp.float32)
        y = y + jnp.dot(xb * keep[:, 1:2].astype(jnp.bfloat16),
                        w[1].astype(jnp.bfloat16),
                        preferred_element_type=jnp.float32)
        acc_ref[...] = y.astype(jnp.bfloat16)

        for k in range(N_STEPS):
            partner = my ^ (1 << k)
            send_ref[...] = acc_ref[...]
            rdma = pltpu.make_async_remote_copy(
                src_ref=send_ref,
                dst_ref=recv_ref.at[k],
                send_sem=send_sems.at[k],
                recv_sem=recv_sems.at[k],
                device_id=(partner,),
                device_id_type=pl.DeviceIdType.MESH,
            )
            rdma.start()
            rdma.wait()
            acc_ref[...] = acc_ref[...] + recv_ref[k]

        out_ref[...] = acc_ref[...].astype(jnp.float32)

    return pl.pallas_call(
        body,
        out_shape=jax.ShapeDtypeStruct((N_TOK, D_OUT), jnp.float32),
        in_specs=[
            pl.BlockSpec(memory_space=pltpu.VMEM),
            pl.BlockSpec(memory_space=pltpu.VMEM),
            pl.BlockSpec(memory_space=pltpu.VMEM),
        ],
        out_specs=pl.BlockSpec(memory_space=pltpu.VMEM),
        scratch_shapes=[
            pltpu.VMEM((N_TOK, D_OUT), jnp.bfloat16),
            pltpu.VMEM((N_TOK, D_OUT), jnp.bfloat16),
            pltpu.VMEM((N_STEPS, N_TOK, D_OUT), jnp.bfloat16),
            pltpu.SemaphoreType.DMA((N_STEPS,)),
            pltpu.SemaphoreType.DMA((N_STEPS,)),
        ],
        compiler_params=pltpu.CompilerParams(collective_id=0),
    )(x, route_idx, expert_W)


# device time: 14695 ns/iter; 1.9464x vs baseline; 1.9464x over previous
import jax
import jax.numpy as jnp
from jax import lax
from jax.experimental import pallas as pl
from jax.experimental.pallas import tpu as pltpu

N_DEV = 32
N_TOK = 256
D_IN = 128
D_OUT = 256
EXP_PER = 2
CAPACITY = 3
N_EXPERTS = 64
SLOTS_PER = 8
N_SLOTS = N_DEV * SLOTS_PER


def kernel(x, router_W, route_idx, expert_W):
    del router_W

    def body(x_ref, ridx_ref, ew_ref, out_ref,
             c_ref, send_ref, send_sems, recv_sems):
        my = lax.axis_index("i")

        barrier_sem = pltpu.get_barrier_semaphore()
        for dd in range(1, N_DEV):
            pl.semaphore_signal(
                barrier_sem, inc=1,
                device_id=(lax.rem(my + dd, N_DEV),),
                device_id_type=pl.DeviceIdType.MESH,
            )

        ridx = ridx_ref[...]
        e0 = my * EXP_PER
        b0 = ridx == e0
        b1 = ridx == e0 + 1
        row = lax.broadcasted_iota(jnp.int32, (N_TOK, N_TOK), 0)
        col = lax.broadcasted_iota(jnp.int32, (N_TOK, N_TOK), 1)
        tri = (row >= col).astype(jnp.float32)
        m2 = jnp.concatenate(
            [b0.astype(jnp.float32), b1.astype(jnp.float32)], axis=1)
        pos2 = jnp.dot(tri, m2, preferred_element_type=jnp.float32)
        p0 = pos2[:, 0:1]
        p1 = pos2[:, 1:2]
        kept0 = b0 & (p0 <= float(CAPACITY))
        kept1 = b1 & (p1 <= float(CAPACITY))

        xb = x_ref[...].astype(jnp.bfloat16)
        w = ew_ref[...]
        y = jnp.dot(xb * kept0.astype(jnp.bfloat16),
                    w[0].astype(jnp.bfloat16),
                    preferred_element_type=jnp.float32)
        y = y + jnp.dot(xb * kept1.astype(jnp.bfloat16),
                        w[1].astype(jnp.bfloat16),
                        preferred_element_type=jnp.float32)
        yb = y.astype(jnp.bfloat16)

        sl = jnp.where(
            kept0, p0.astype(jnp.int32) - 1,
            jnp.where(kept1, p1.astype(jnp.int32) + (CAPACITY - 1), -1))
        qm = (sl == lax.broadcasted_iota(jnp.int32, (N_TOK, SLOTS_PER), 1)
              ).astype(jnp.bfloat16)
        c_mine = lax.dot_general(
            qm, yb, (((0,), (0,)), ((), ())),
            preferred_element_type=jnp.float32,
        ).astype(jnp.bfloat16)
        send_ref[...] = c_mine
        c_ref[pl.ds(my * SLOTS_PER, SLOTS_PER), :] = c_mine

        pl.semaphore_wait(barrier_sem, N_DEV - 1)
        sends = []
        for dd in range(1, N_DEV):
            tgt = lax.rem(my + dd, N_DEV)
            rdma = pltpu.make_async_remote_copy(
                src_ref=send_ref,
                dst_ref=c_ref.at[pl.ds(my * SLOTS_PER, SLOTS_PER)],
                send_sem=send_sems.at[dd - 1],
                recv_sem=recv_sems.at[dd - 1],
                device_id=(tgt,),
                device_id_type=pl.DeviceIdType.MESH,
            )
            rdma.start()
            sends.append(rdma)

        m_exp = (ridx == lax.broadcasted_iota(jnp.int32, (N_TOK, N_EXPERTS), 1)
                 ).astype(jnp.float32)
        pos = jnp.dot(tri, m_exp, preferred_element_type=jnp.float32)
        posk = jnp.sum(m_exp * pos, axis=1, keepdims=True).astype(jnp.int32)
        kept = posk <= CAPACITY
        g = jnp.where(
            kept,
            lax.div(ridx, EXP_PER) * SLOTS_PER
            + lax.rem(ridx, EXP_PER) * CAPACITY + posk - 1,
            -1,
        )
        q = (g == lax.broadcasted_iota(jnp.int32, (N_TOK, N_SLOTS), 1)
             ).astype(jnp.bfloat16)

        for dd in range(1, N_DEV):
            src_dev = lax.rem(my + (N_DEV - dd), N_DEV)
            recv = pltpu.make_async_remote_copy(
                src_ref=send_ref,
                dst_ref=c_ref.at[pl.ds(src_dev * SLOTS_PER, SLOTS_PER)],
                send_sem=send_sems.at[dd - 1],
                recv_sem=recv_sems.at[dd - 1],
                device_id=(src_dev,),
                device_id_type=pl.DeviceIdType.MESH,
            )
            recv.wait_recv()

        out_ref[...] = jnp.dot(q, c_ref[...],
                               preferred_element_type=jnp.float32)

        for rdma in sends:
            rdma.wait_send()

    return pl.pallas_call(
        body,
        out_shape=jax.ShapeDtypeStruct((N_TOK, D_OUT), jnp.float32),
        in_specs=[
            pl.BlockSpec(memory_space=pltpu.VMEM),
            pl.BlockSpec(memory_space=pltpu.VMEM),
            pl.BlockSpec(memory_space=pltpu.VMEM),
        ],
        out_specs=pl.BlockSpec(memory_space=pltpu.VMEM),
        scratch_shapes=[
            pltpu.VMEM((N_SLOTS, D_OUT), jnp.bfloat16),
            pltpu.VMEM((SLOTS_PER, D_OUT), jnp.bfloat16),
            pltpu.SemaphoreType.DMA((N_DEV - 1,)),
            pltpu.SemaphoreType.DMA((N_DEV - 1,)),
        ],
        compiler_params=pltpu.CompilerParams(collective_id=0),
    )(x, route_idx, expert_W)


# device time: 14660 ns/iter; 1.9510x vs baseline; 1.0024x over previous
import jax
import jax.numpy as jnp
from jax import lax
from jax.experimental import pallas as pl
from jax.experimental.pallas import tpu as pltpu

N_DEV = 32
N_TOK = 256
D_IN = 128
D_OUT = 256
EXP_PER = 2
CAPACITY = 3
N_EXPERTS = 64
SLOTS_PER = 8
N_SLOTS = N_DEV * SLOTS_PER


def kernel(x, router_W, route_idx, expert_W):
    del router_W

    def body(x_ref, ridx_ref, ew_ref, out_ref,
             c_ref, send_ref, send_sems, recv_sems):
        my = lax.axis_index("i")

        barrier_sem = pltpu.get_barrier_semaphore()
        for dd in range(1, N_DEV):
            pl.semaphore_signal(
                barrier_sem, inc=1,
                device_id=(lax.rem(my + dd, N_DEV),),
                device_id_type=pl.DeviceIdType.MESH,
            )

        row = lax.broadcasted_iota(jnp.int32, (N_TOK, N_TOK), 0)
        col = lax.broadcasted_iota(jnp.int32, (N_TOK, N_TOK), 1)
        tri = (row >= col).astype(jnp.float32)
        ridx = ridx_ref[...]
        e0 = my * EXP_PER
        b0 = ridx == e0
        b1 = ridx == e0 + 1
        m2 = jnp.concatenate(
            [b0.astype(jnp.float32), b1.astype(jnp.float32)], axis=1)
        pos2 = jnp.dot(tri, m2, preferred_element_type=jnp.float32)
        p0 = pos2[:, 0:1]
        p1 = pos2[:, 1:2]
        kept0 = b0 & (p0 <= float(CAPACITY))
        kept1 = b1 & (p1 <= float(CAPACITY))

        xb = x_ref[...].astype(jnp.bfloat16)
        w = ew_ref[...]
        y = jnp.dot(xb * kept0.astype(jnp.bfloat16),
                    w[0].astype(jnp.bfloat16),
                    preferred_element_type=jnp.float32)
        y = y + jnp.dot(xb * kept1.astype(jnp.bfloat16),
                        w[1].astype(jnp.bfloat16),
                        preferred_element_type=jnp.float32)
        yb = y.astype(jnp.bfloat16)

        sl = jnp.where(
            kept0, p0.astype(jnp.int32) - 1,
            jnp.where(kept1, p1.astype(jnp.int32) + (CAPACITY - 1), -1))
        qm = (sl == lax.broadcasted_iota(jnp.int32, (N_TOK, SLOTS_PER), 1)
              ).astype(jnp.bfloat16)
        c_mine = lax.dot_general(
            qm, yb, (((0,), (0,)), ((), ())),
            preferred_element_type=jnp.float32,
        ).astype(jnp.bfloat16)
        send_ref[...] = c_mine
        c_ref[pl.ds(my * SLOTS_PER, SLOTS_PER), :] = c_mine

        pl.semaphore_wait(barrier_sem, N_DEV - 1)
        sends = []
        for dd in range(1, N_DEV):
            tgt = lax.rem(my + dd, N_DEV)
            rdma = pltpu.make_async_remote_copy(
                src_ref=send_ref,
                dst_ref=c_ref.at[pl.ds(my * SLOTS_PER, SLOTS_PER)],
                send_sem=send_sems.at[dd - 1],
                recv_sem=recv_sems.at[dd - 1],
                device_id=(tgt,),
                device_id_type=pl.DeviceIdType.MESH,
            )
            rdma.start()
            sends.append(rdma)

        m_exp = (ridx == lax.broadcasted_iota(jnp.int32, (N_TOK, N_EXPERTS), 1)
                 ).astype(jnp.float32)
        pos = jnp.dot(tri, m_exp, preferred_element_type=jnp.float32)
        posk = jnp.sum(m_exp * pos, axis=1, keepdims=True).astype(jnp.int32)
        kept = posk <= CAPACITY
        g = jnp.where(
            kept,
            lax.div(ridx, EXP_PER) * SLOTS_PER
            + lax.rem(ridx, EXP_PER) * CAPACITY + posk - 1,
            -1,
        )
        q = (g == lax.broadcasted_iota(jnp.int32, (N_TOK, N_SLOTS), 1)
             ).astype(jnp.bfloat16)

        for dd in range(1, N_DEV):
            src_dev = lax.rem(my + (N_DEV - dd), N_DEV)
            recv = pltpu.make_async_remote_copy(
                src_ref=send_ref,
                dst_ref=c_ref.at[pl.ds(src_dev * SLOTS_PER, SLOTS_PER)],
                send_sem=send_sems.at[dd - 1],
                recv_sem=recv_sems.at[dd - 1],
                device_id=(src_dev,),
                device_id_type=pl.DeviceIdType.MESH,
            )
            recv.wait_recv()

        out_ref[...] = jnp.dot(q, c_ref[...],
                               preferred_element_type=jnp.float32
                               ).astype(jnp.bfloat16)

        for rdma in sends:
            rdma.wait_send()

    return pl.pallas_call(
        body,
        out_shape=jax.ShapeDtypeStruct((N_TOK, D_OUT), jnp.bfloat16),
        in_specs=[
            pl.BlockSpec(memory_space=pltpu.VMEM),
            pl.BlockSpec(memory_space=pltpu.VMEM),
            pl.BlockSpec(memory_space=pltpu.VMEM),
        ],
        out_specs=pl.BlockSpec(memory_space=pltpu.VMEM),
        scratch_shapes=[
            pltpu.VMEM((N_SLOTS, D_OUT), jnp.bfloat16),
            pltpu.VMEM((SLOTS_PER, D_OUT), jnp.bfloat16),
            pltpu.SemaphoreType.DMA((N_DEV - 1,)),
            pltpu.SemaphoreType.DMA((N_DEV - 1,)),
        ],
        compiler_params=pltpu.CompilerParams(collective_id=0),
    )(x, route_idx, expert_W)
